# Initial kernel scaffold; baseline (speedup 1.0000x reference)
import jax
import jax.numpy as jnp
from jax.experimental import pallas as pl


def kernel(x, edge_index, batch, W1, b1, g1, be1, W2, b2, g2, be2, Wf1, bf1, Wf2, bf2):
    raise NotImplementedError("write your pallas kernel here")


# speedup vs baseline: 3.7511x; 3.7511x over previous
import functools

import jax
import jax.numpy as jnp
from jax import lax
from jax.experimental import pallas as pl
from jax.experimental.pallas import tpu as pltpu
from jax.experimental.pallas import tpu_sc as plsc

N = 10000
E = 160000
D = 256
H = 128
NC = 2
NS = 16

EB = 40
EPT_DEG = E // (NC * NS)
NB_DEG = EPT_DEG // EB
INIT_CH = 624


def _deg_body(dst_hbm, out_hbm, idx_v, ones_v, buf_v, acc_sh):
    c = lax.axis_index("c")
    s = lax.axis_index("s")
    for j in range(48 // 16):
        ones_v[pl.ds(j * 16, 16)] = jnp.full((16,), 1.0, jnp.float32)
    for j in range(INIT_CH // 16):
        buf_v[pl.ds(j * 16, 16)] = jnp.full((16,), 0.0, jnp.float32)
    pltpu.sync_copy(buf_v, acc_sh.at[pl.ds(s * INIT_CH, INIT_CH)])

    @pl.when(s == 0)
    def _():
        pltpu.sync_copy(buf_v.at[pl.ds(0, 16)],
                        acc_sh.at[pl.ds(NS * INIT_CH, 16)])

    plsc.subcore_barrier()
    edge0 = (c * NS + s) * EPT_DEG

    def body(i, carry):
        base = pl.multiple_of(edge0 + i * EB, 8)
        pltpu.sync_copy(dst_hbm.at[pl.ds(base, EB)], idx_v)
        pltpu.sync_copy(ones_v.at[pl.ds(0, EB)], acc_sh.at[idx_v], add=True)
        return carry

    lax.fori_loop(0, NB_DEG, body, 0)
    plsc.subcore_barrier()
    pltpu.sync_copy(acc_sh.at[pl.ds(s * INIT_CH, INIT_CH)], buf_v)
    pltpu.sync_copy(buf_v, out_hbm.at[pl.ds(c * N + s * INIT_CH, INIT_CH)])

    @pl.when(s == 0)
    def _():
        pltpu.sync_copy(acc_sh.at[pl.ds(NS * INIT_CH, 16)],
                        buf_v.at[pl.ds(0, 16)])
        pltpu.sync_copy(buf_v.at[pl.ds(0, 16)],
                        out_hbm.at[pl.ds(c * N + NS * INIT_CH, 16)])


def _deg_call(dst):
    f = pl.kernel(
        _deg_body,
        out_type=jax.ShapeDtypeStruct((NC * N,), jnp.float32),
        mesh=plsc.VectorSubcoreMesh(core_axis_name="c", subcore_axis_name="s",
                                    num_cores=NC, num_subcores=NS),
        scratch_types=[
            pltpu.VMEM((EB,), jnp.int32),
            pltpu.VMEM((48,), jnp.float32),
            pltpu.VMEM((INIT_CH,), jnp.float32),
            pltpu.VMEM_SHARED((N,), jnp.float32),
        ],
    )
    return f(dst)


B = 80
EPT = E // NS
NB = EPT // B
NH = N // 2
ACC_R = NH + 8
RPT = 312
RTAIL = NH - NS * RPT


def _agg_body(hp_hbm, src_hbm, dst_hbm, out_hbm,
              idx_s, idx_d, rows_v, buf_v, acc_sh, sem):
    c = lax.axis_index("c")
    s = lax.axis_index("s")
    coff = c * N
    edge0 = s * EPT
    for p in range(2):
        nbase = p * NH
        row0 = coff + nbase + s * RPT
        pltpu.sync_copy(hp_hbm.at[pl.ds(row0, RPT)], buf_v)
        pltpu.sync_copy(buf_v, acc_sh.at[pl.ds(s * RPT, RPT)])

        @pl.when(s == 0)
        def _():
            pltpu.sync_copy(hp_hbm.at[pl.ds(coff + nbase + NS * RPT, RTAIL)],
                            buf_v.at[pl.ds(0, RTAIL)])
            pltpu.sync_copy(buf_v.at[pl.ds(0, RTAIL)],
                            acc_sh.at[pl.ds(NS * RPT, RTAIL)])

        plsc.subcore_barrier()

        def body(i, carry):
            base = pl.multiple_of(edge0 + i * B, 8)
            pltpu.sync_copy(src_hbm.at[pl.ds(base, B)], idx_s)
            for j in range(B // 16):
                sl = pl.ds(j * 16, 16)
                idx_s[sl] = idx_s[sl] + coff
            pltpu.async_copy(hp_hbm.at[idx_s], rows_v, sem).wait()
            pltpu.sync_copy(dst_hbm.at[pl.ds(base, B)], idx_d)
            for j in range(B // 16):
                sl = pl.ds(j * 16, 16)
                v = idx_d[sl] - nbase
                ok = (v >= 0) & (v < NH)
                idx_d[sl] = jnp.where(ok, v, NH)
            pltpu.sync_copy(rows_v, acc_sh.at[idx_d], add=True)
            return carry

        lax.fori_loop(0, NB, body, 0)
        plsc.subcore_barrier()
        pltpu.sync_copy(acc_sh.at[pl.ds(s * RPT, RPT)], buf_v)
        pltpu.sync_copy(buf_v, out_hbm.at[pl.ds(row0, RPT)])

        @pl.when(s == 0)
        def _():
            pltpu.sync_copy(acc_sh.at[pl.ds(NS * RPT, RTAIL)],
                            buf_v.at[pl.ds(0, RTAIL)])
            pltpu.sync_copy(buf_v.at[pl.ds(0, RTAIL)],
                            out_hbm.at[pl.ds(coff + nbase + NS * RPT, RTAIL)])

        plsc.subcore_barrier()


@functools.cache
def _agg_kernel():
    return pl.kernel(
        _agg_body,
        out_type=jax.ShapeDtypeStruct((NC * N, H), jnp.float32),
        mesh=plsc.VectorSubcoreMesh(core_axis_name="c", subcore_axis_name="s",
                                    num_cores=NC, num_subcores=NS),
        scratch_types=[
            pltpu.VMEM((B,), jnp.int32),
            pltpu.VMEM((B,), jnp.int32),
            pltpu.VMEM((B, H), jnp.float32),
            pltpu.VMEM((RPT, H), jnp.float32),
            pltpu.VMEM_SHARED((ACC_R, H), jnp.float32),
            pltpu.SemaphoreType.DMA,
        ],
    )


def _agg_call(hp2d, src, dst):
    return _agg_kernel()(hp2d, src, dst)


R = 1000
_HI = lax.Precision.HIGHEST


def _dis_from(deg_ref):
    d = deg_ref[:, 0] + deg_ref[:, 1] + 1.0
    return lax.rsqrt(d)[:, None]


def _mm1_body(x_ref, w_ref, deg_ref, out_ref):
    dis = _dis_from(deg_ref)
    out_ref[0] = jnp.dot(x_ref[...], w_ref[...],
                         preferred_element_type=jnp.float32,
                         precision=_HI) * dis


def _mm1_call(x, W1, degT):
    return pl.pallas_call(
        _mm1_body,
        grid=(NC, N // R),
        in_specs=[
            pl.BlockSpec((R, D), lambda c, i: (i, 0)),
            pl.BlockSpec((D, H), lambda c, i: (0, c)),
            pl.BlockSpec((R, 2), lambda c, i: (i, 0)),
        ],
        out_specs=pl.BlockSpec((1, R, H), lambda c, i: (c, i, 0)),
        out_shape=jax.ShapeDtypeStruct((NC, N, H), jnp.float32),
    )(x, W1, degT)


def _ln_relu(agg_ref, dis, b_ref, g_ref, be_ref):
    v0 = agg_ref[0] * dis + b_ref[0]
    v1 = agg_ref[1] * dis + b_ref[1]
    mu = (jnp.sum(v0, axis=1) + jnp.sum(v1, axis=1)) * (1.0 / D)
    c0 = v0 - mu[:, None]
    c1 = v1 - mu[:, None]
    var = (jnp.sum(c0 * c0, axis=1) + jnp.sum(c1 * c1, axis=1)) * (1.0 / D)
    inv = lax.rsqrt(var + 1e-5)[:, None]
    t0 = jnp.maximum(c0 * inv * g_ref[0] + be_ref[0], 0.0)
    t1 = jnp.maximum(c1 * inv * g_ref[1] + be_ref[1], 0.0)
    return jnp.concatenate([t0, t1], axis=1)


def _ln_mm_body(agg_ref, deg_ref, b_ref, g_ref, be_ref, w_ref, out_ref):
    dis = _dis_from(deg_ref)
    t = _ln_relu(agg_ref, dis, b_ref, g_ref, be_ref)
    out_ref[0] = jnp.dot(t, w_ref[...],
                         preferred_element_type=jnp.float32,
                         precision=_HI) * dis


def _ln_mm_call(agg, degT, b, g, be, W2):
    full = lambda c, i: (0, 0)
    return pl.pallas_call(
        _ln_mm_body,
        grid=(NC, N // R),
        in_specs=[
            pl.BlockSpec((NC, R, H), lambda c, i: (0, i, 0)),
            pl.BlockSpec((R, 2), lambda c, i: (i, 0)),
            pl.BlockSpec((NC, H), full),
            pl.BlockSpec((NC, H), full),
            pl.BlockSpec((NC, H), full),
            pl.BlockSpec((D, H), lambda c, i: (0, c)),
        ],
        out_specs=pl.BlockSpec((1, R, H), lambda c, i: (c, i, 0)),
        out_shape=jax.ShapeDtypeStruct((NC, N, H), jnp.float32),
    )(agg, degT, b, g, be, W2)


def _head_body(agg_ref, deg_ref, b_ref, g_ref, be_ref,
               wf1_ref, bf1_ref, wf2_ref, bf2_ref, out_ref):
    dis = _dis_from(deg_ref)
    t = _ln_relu(agg_ref, dis, b_ref, g_ref, be_ref)
    u = jnp.maximum(jnp.dot(t, wf1_ref[...],
                            preferred_element_type=jnp.float32,
                            precision=_HI) + bf1_ref[0], 0.0)
    out_ref[...] = jnp.tanh(jnp.dot(u, wf2_ref[...],
                                    preferred_element_type=jnp.float32,
                                    precision=_HI) + bf2_ref[0])


def _head_call(agg, degT, b, g, be, Wf1, bf1, Wf2p, bf2p):
    full = lambda i: (0, 0)
    return pl.pallas_call(
        _head_body,
        grid=(N // R,),
        in_specs=[
            pl.BlockSpec((NC, R, H), lambda i: (0, i, 0)),
            pl.BlockSpec((R, 2), lambda i: (i, 0)),
            pl.BlockSpec((NC, H), full),
            pl.BlockSpec((NC, H), full),
            pl.BlockSpec((NC, H), full),
            pl.BlockSpec((D, H), full),
            pl.BlockSpec((1, H), full),
            pl.BlockSpec((H, H), full),
            pl.BlockSpec((1, H), full),
        ],
        out_specs=pl.BlockSpec((R, H), lambda i: (i, 0)),
        out_shape=jax.ShapeDtypeStruct((N, H), jnp.float32),
    )(agg, degT, b, g, be, Wf1, bf1, Wf2p, bf2p)


def kernel(x, edge_index, batch, W1, b1, g1, be1, W2, b2, g2, be2,
           Wf1, bf1, Wf2, bf2):
    src = edge_index[0]
    dst = edge_index[1]
    deg2 = _deg_call(dst).reshape(NC, N)
    degT = jnp.transpose(deg2)

    hp1 = _mm1_call(x, W1, degT)

    W_st = jnp.stack([W2, W2])
    b_st = jnp.stack([b1.reshape(NC, H), b2.reshape(NC, H)])
    g_st = jnp.stack([g1.reshape(NC, H), g2.reshape(NC, H)])
    be_st = jnp.stack([be1.reshape(NC, H), be2.reshape(NC, H)])

    def step(hp, xs):
        Wl, bl, gl, bel = xs
        agg = _agg_call(hp.reshape(NC * N, H), src, dst).reshape(NC, N, H)
        hp_next = _ln_mm_call(agg, degT, bl, gl, bel, Wl)
        return hp_next, agg

    _, aggs = lax.scan(step, hp1, (W_st, b_st, g_st, be_st))
    agg2 = aggs[1]

    Wf2p = jnp.pad(Wf2, ((0, 0), (0, H - Wf2.shape[1])))
    bf2p = jnp.pad(bf2, (0, H - bf2.shape[0])).reshape(1, H)
    o = _head_call(agg2, degT,
                   b2.reshape(NC, H), g2.reshape(NC, H), be2.reshape(NC, H),
                   Wf1, bf1.reshape(1, H), Wf2p, bf2p)
    return o[:, :Wf2.shape[1]]


# speedup vs baseline: 6.7991x; 1.8126x over previous
import functools

import jax
import jax.numpy as jnp
from jax import lax
from jax.experimental import pallas as pl
from jax.experimental.pallas import tpu as pltpu
from jax.experimental.pallas import tpu_sc as plsc

N = 10000
E = 160000
D = 256
H = 128
NC = 2
NS = 16

EB = 128
EPT_DEG = 4992
INIT_CH = 624


def _deg_body(dst_hbm, out_hbm, idx_v, ones_v, buf_v, acc_sh):
    c = lax.axis_index("c")
    s = lax.axis_index("s")
    for j in range(EB // 16):
        ones_v[pl.ds(j * 16, 16)] = jnp.full((16,), 1.0, jnp.float32)
    for j in range(INIT_CH // 16):
        buf_v[pl.ds(j * 16, 16)] = jnp.full((16,), 0.0, jnp.float32)
    pltpu.sync_copy(buf_v, acc_sh.at[pl.ds(s * INIT_CH, INIT_CH)])

    @pl.when(s == 0)
    def _():
        pltpu.sync_copy(buf_v.at[pl.ds(0, 16)],
                        acc_sh.at[pl.ds(NS * INIT_CH, 16)])

    plsc.subcore_barrier()
    edge0 = c * (E // NC) + s * EPT_DEG
    nb = 39 + jnp.where(s == NS - 1, 1, 0)

    def body(i, carry):
        base = pl.multiple_of(edge0 + i * EB, 8)
        pltpu.sync_copy(dst_hbm.at[pl.ds(base, EB)], idx_v)
        pltpu.sync_copy(ones_v, acc_sh.at[idx_v], add=True)
        return carry

    lax.fori_loop(0, nb, body, 0)
    plsc.subcore_barrier()
    pltpu.sync_copy(acc_sh.at[pl.ds(s * INIT_CH, INIT_CH)], buf_v)
    pltpu.sync_copy(buf_v, out_hbm.at[pl.ds(c * N + s * INIT_CH, INIT_CH)])

    @pl.when(s == 0)
    def _():
        pltpu.sync_copy(acc_sh.at[pl.ds(NS * INIT_CH, 16)],
                        buf_v.at[pl.ds(0, 16)])
        pltpu.sync_copy(buf_v.at[pl.ds(0, 16)],
                        out_hbm.at[pl.ds(c * N + NS * INIT_CH, 16)])


def _deg_call(dst):
    f = pl.kernel(
        _deg_body,
        out_type=jax.ShapeDtypeStruct((NC * N,), jnp.float32),
        mesh=plsc.VectorSubcoreMesh(core_axis_name="c", subcore_axis_name="s",
                                    num_cores=NC, num_subcores=NS),
        scratch_types=[
            pltpu.VMEM((EB,), jnp.int32),
            pltpu.VMEM((EB,), jnp.float32),
            pltpu.VMEM((INIT_CH,), jnp.float32),
            pltpu.VMEM_SHARED((N,), jnp.float32),
        ],
    )
    return f(dst)


B = 128
EPT = 9984
NH = N // 2
ACC_R = NH + 8
RPT = 312
RTAIL = NH - NS * RPT


def _agg_body(hp_hbm, src_hbm, dst_hbm, out_hbm,
              idx_s0, idx_s1, idx_d0, idx_d1, rows0, rows1, buf_v, acc_sh,
              gsem0, gsem1, ssem0, ssem1):
    c = lax.axis_index("c")
    s = lax.axis_index("s")
    coff = c * N
    edge0 = s * EPT
    nedges = EPT + jnp.where(s == NS - 1, 2 * B, 0)
    max_base = edge0 + nedges - B
    nb2 = (39 + jnp.where(s == NS - 1, 1, 0)).astype(jnp.int32)
    idx_s = (idx_s0, idx_s1)
    idx_d = (idx_d0, idx_d1)
    rows = (rows0, rows1)
    gsem = (gsem0, gsem1)
    ssem = (ssem0, ssem1)

    def load_src(b, j):
        base = pl.multiple_of(jnp.minimum(edge0 + b * B, max_base), 8)
        pltpu.sync_copy(src_hbm.at[pl.ds(base, B)], idx_s[j])
        for k in range(B // 16):
            sl = pl.ds(k * 16, 16)
            idx_s[j][sl] = idx_s[j][sl] + coff

    for p in range(2):
        nbase = p * NH
        row0 = coff + nbase + s * RPT
        pltpu.sync_copy(hp_hbm.at[pl.ds(row0, RPT)], buf_v)
        pltpu.sync_copy(buf_v, acc_sh.at[pl.ds(s * RPT, RPT)])

        @pl.when(s == 0)
        def _():
            pltpu.sync_copy(hp_hbm.at[pl.ds(coff + nbase + NS * RPT, RTAIL)],
                            buf_v.at[pl.ds(0, RTAIL)])
            pltpu.sync_copy(buf_v.at[pl.ds(0, RTAIL)],
                            acc_sh.at[pl.ds(NS * RPT, RTAIL)])

        plsc.subcore_barrier()

        for j in range(2):
            load_src(jnp.int32(j), j)
            pltpu.async_copy(hp_hbm.at[idx_s[j]], rows[j], gsem[j])

        def body(k, carry):
            for j in range(2):
                b = 2 * k + j
                pltpu.make_async_copy(hp_hbm.at[idx_s[j]], rows[j],
                                      gsem[j]).wait()
                bb = pl.multiple_of(jnp.minimum(edge0 + b * B, max_base), 8)
                pltpu.sync_copy(dst_hbm.at[pl.ds(bb, B)], idx_d[j])
                for t in range(B // 16):
                    sl = pl.ds(t * 16, 16)
                    v = idx_d[j][sl] - nbase
                    ok = (v >= 0) & (v < NH)
                    idx_d[j][sl] = jnp.where(ok, v, NH)
                pltpu.async_copy(rows[j], acc_sh.at[idx_d[j]], ssem[j],
                                 add=True)
                load_src(b + 2, j)
                pltpu.make_async_copy(rows[j], acc_sh.at[idx_d[j]],
                                      ssem[j]).wait()
                pltpu.async_copy(hp_hbm.at[idx_s[j]], rows[j], gsem[j])
            return carry

        lax.fori_loop(0, nb2, body, 0)
        for j in range(2):
            pltpu.make_async_copy(hp_hbm.at[idx_s[j]], rows[j],
                                  gsem[j]).wait()
        plsc.subcore_barrier()
        pltpu.sync_copy(acc_sh.at[pl.ds(s * RPT, RPT)], buf_v)
        pltpu.sync_copy(buf_v, out_hbm.at[pl.ds(row0, RPT)])

        @pl.when(s == 0)
        def _():
            pltpu.sync_copy(acc_sh.at[pl.ds(NS * RPT, RTAIL)],
                            buf_v.at[pl.ds(0, RTAIL)])
            pltpu.sync_copy(buf_v.at[pl.ds(0, RTAIL)],
                            out_hbm.at[pl.ds(coff + nbase + NS * RPT, RTAIL)])

        plsc.subcore_barrier()


@functools.cache
def _agg_kernel():
    return pl.kernel(
        _agg_body,
        out_type=jax.ShapeDtypeStruct((NC * N, H), jnp.float32),
        mesh=plsc.VectorSubcoreMesh(core_axis_name="c", subcore_axis_name="s",
                                    num_cores=NC, num_subcores=NS),
        scratch_types=[
            pltpu.VMEM((B,), jnp.int32),
            pltpu.VMEM((B,), jnp.int32),
            pltpu.VMEM((B,), jnp.int32),
            pltpu.VMEM((B,), jnp.int32),
            pltpu.VMEM((B, H), jnp.float32),
            pltpu.VMEM((B, H), jnp.float32),
            pltpu.VMEM((RPT, H), jnp.float32),
            pltpu.VMEM_SHARED((ACC_R, H), jnp.float32),
            pltpu.SemaphoreType.DMA,
            pltpu.SemaphoreType.DMA,
            pltpu.SemaphoreType.DMA,
            pltpu.SemaphoreType.DMA,
        ],
    )


def _agg_call(hp2d, src, dst):
    return _agg_kernel()(hp2d, src, dst)


R = 1000
_HI = lax.Precision.HIGHEST


def _dis_from(deg_ref):
    d = deg_ref[:, 0] + deg_ref[:, 1] + 1.0
    return lax.rsqrt(d)[:, None]


def _mm1_body(x_ref, w_ref, deg_ref, out_ref):
    dis = _dis_from(deg_ref)
    out_ref[0] = jnp.dot(x_ref[...], w_ref[...],
                         preferred_element_type=jnp.float32,
                         precision=_HI) * dis


def _mm1_call(x, W1, degT):
    return pl.pallas_call(
        _mm1_body,
        grid=(NC, N // R),
        in_specs=[
            pl.BlockSpec((R, D), lambda c, i: (i, 0)),
            pl.BlockSpec((D, H), lambda c, i: (0, c)),
            pl.BlockSpec((R, 2), lambda c, i: (i, 0)),
        ],
        out_specs=pl.BlockSpec((1, R, H), lambda c, i: (c, i, 0)),
        out_shape=jax.ShapeDtypeStruct((NC, N, H), jnp.float32),
    )(x, W1, degT)


def _ln_relu(agg_ref, dis, b_ref, g_ref, be_ref):
    v0 = agg_ref[0] * dis + b_ref[0]
    v1 = agg_ref[1] * dis + b_ref[1]
    mu = (jnp.sum(v0, axis=1) + jnp.sum(v1, axis=1)) * (1.0 / D)
    c0 = v0 - mu[:, None]
    c1 = v1 - mu[:, None]
    var = (jnp.sum(c0 * c0, axis=1) + jnp.sum(c1 * c1, axis=1)) * (1.0 / D)
    inv = lax.rsqrt(var + 1e-5)[:, None]
    t0 = jnp.maximum(c0 * inv * g_ref[0] + be_ref[0], 0.0)
    t1 = jnp.maximum(c1 * inv * g_ref[1] + be_ref[1], 0.0)
    return jnp.concatenate([t0, t1], axis=1)


def _ln_mm_body(agg_ref, deg_ref, b_ref, g_ref, be_ref, w_ref, out_ref):
    dis = _dis_from(deg_ref)
    t = _ln_relu(agg_ref, dis, b_ref, g_ref, be_ref)
    out_ref[0] = jnp.dot(t, w_ref[...],
                         preferred_element_type=jnp.float32,
                         precision=_HI) * dis


def _ln_mm_call(agg, degT, b, g, be, W2):
    full = lambda c, i: (0, 0)
    return pl.pallas_call(
        _ln_mm_body,
        grid=(NC, N // R),
        in_specs=[
            pl.BlockSpec((NC, R, H), lambda c, i: (0, i, 0)),
            pl.BlockSpec((R, 2), lambda c, i: (i, 0)),
            pl.BlockSpec((NC, H), full),
            pl.BlockSpec((NC, H), full),
            pl.BlockSpec((NC, H), full),
            pl.BlockSpec((D, H), lambda c, i: (0, c)),
        ],
        out_specs=pl.BlockSpec((1, R, H), lambda c, i: (c, i, 0)),
        out_shape=jax.ShapeDtypeStruct((NC, N, H), jnp.float32),
    )(agg, degT, b, g, be, W2)


def _head_body(agg_ref, deg_ref, b_ref, g_ref, be_ref,
               wf1_ref, bf1_ref, wf2_ref, bf2_ref, out_ref):
    dis = _dis_from(deg_ref)
    t = _ln_relu(agg_ref, dis, b_ref, g_ref, be_ref)
    u = jnp.maximum(jnp.dot(t, wf1_ref[...],
                            preferred_element_type=jnp.float32,
                            precision=_HI) + bf1_ref[0], 0.0)
    out_ref[...] = jnp.tanh(jnp.dot(u, wf2_ref[...],
                                    preferred_element_type=jnp.float32,
                                    precision=_HI) + bf2_ref[0])


def _head_call(agg, degT, b, g, be, Wf1, bf1, Wf2p, bf2p):
    full = lambda i: (0, 0)
    return pl.pallas_call(
        _head_body,
        grid=(N // R,),
        in_specs=[
            pl.BlockSpec((NC, R, H), lambda i: (0, i, 0)),
            pl.BlockSpec((R, 2), lambda i: (i, 0)),
            pl.BlockSpec((NC, H), full),
            pl.BlockSpec((NC, H), full),
            pl.BlockSpec((NC, H), full),
            pl.BlockSpec((D, H), full),
            pl.BlockSpec((1, H), full),
            pl.BlockSpec((H, H), full),
            pl.BlockSpec((1, H), full),
        ],
        out_specs=pl.BlockSpec((R, H), lambda i: (i, 0)),
        out_shape=jax.ShapeDtypeStruct((N, H), jnp.float32),
    )(agg, degT, b, g, be, Wf1, bf1, Wf2p, bf2p)


def kernel(x, edge_index, batch, W1, b1, g1, be1, W2, b2, g2, be2,
           Wf1, bf1, Wf2, bf2):
    src = edge_index[0]
    dst = edge_index[1]
    deg2 = _deg_call(dst).reshape(NC, N)
    degT = jnp.transpose(deg2)

    hp1 = _mm1_call(x, W1, degT)

    W_st = jnp.stack([W2, W2])
    b_st = jnp.stack([b1.reshape(NC, H), b2.reshape(NC, H)])
    g_st = jnp.stack([g1.reshape(NC, H), g2.reshape(NC, H)])
    be_st = jnp.stack([be1.reshape(NC, H), be2.reshape(NC, H)])

    def step(hp, xs):
        Wl, bl, gl, bel = xs
        agg = _agg_call(hp.reshape(NC * N, H), src, dst).reshape(NC, N, H)
        hp_next = _ln_mm_call(agg, degT, bl, gl, bel, Wl)
        return hp_next, agg

    _, aggs = lax.scan(step, hp1, (W_st, b_st, g_st, be_st))
    agg2 = aggs[1]

    Wf2p = jnp.pad(Wf2, ((0, 0), (0, H - Wf2.shape[1])))
    bf2p = jnp.pad(bf2, (0, H - bf2.shape[0])).reshape(1, H)
    o = _head_call(agg2, degT,
                   b2.reshape(NC, H), g2.reshape(NC, H), be2.reshape(NC, H),
                   Wf1, bf1.reshape(1, H), Wf2p, bf2p)
    return o[:, :Wf2.shape[1]]
